# Initial kernel scaffold; baseline (speedup 1.0000x reference)
import jax
import jax.numpy as jnp
from jax.experimental import pallas as pl


def kernel(z):
    raise NotImplementedError("write your pallas kernel here")


# speedup vs baseline: 27.8228x; 27.8228x over previous
import jax
import jax.numpy as jnp
from jax.experimental import pallas as pl
from jax.experimental.pallas import tpu as pltpu

_T = 0.1
_K = 5
_N = 4096
_D = 128
_BLK = 256


def _and_loss_kernel(z_ref, out_ref):
    i = pl.program_id(0)
    z = z_ref[...]
    nrm = jnp.sqrt(jnp.sum(z * z, axis=1, keepdims=True))
    zn = z / jnp.maximum(nrm, 1e-12)
    zb_raw = z_ref[pl.ds(i * _BLK, _BLK), :]
    bnrm = jnp.sqrt(jnp.sum(zb_raw * zb_raw, axis=1, keepdims=True))
    zb = zb_raw / jnp.maximum(bnrm, 1e-12)

    sim = jnp.dot(zb, zn.T, preferred_element_type=jnp.float32)
    logits = sim * (1.0 / _T)

    row = jax.lax.broadcasted_iota(jnp.int32, (_BLK, _N), 0) + i * _BLK
    col = jax.lax.broadcasted_iota(jnp.int32, (_BLK, _N), 1)
    self_mask = row == col

    masked = jnp.where(self_mask, -1e9, logits)
    m = jnp.max(masked, axis=1, keepdims=True)
    lse = m + jnp.log(jnp.sum(jnp.exp(masked - m), axis=1, keepdims=True))

    cur = logits
    vals = []
    for _ in range(_K + 1):
        v = jnp.max(cur, axis=1, keepdims=True)
        vals.append(v)
        am = jnp.argmax(cur, axis=1)
        cur = jnp.where(col == am[:, None], -jnp.inf, cur)

    acc = jnp.ones_like(vals[1])
    for t in range(2, _K + 1):
        acc = acc + jnp.exp(vals[t] - vals[1])
    num = vals[1] + jnp.log(acc)

    part = jnp.sum(num - lse, keepdims=True)

    @pl.when(i == 0)
    def _():
        out_ref[...] = jnp.zeros((1, 1), jnp.float32)

    out_ref[...] += part


def kernel(z):
    partial = pl.pallas_call(
        _and_loss_kernel,
        grid=(_N // _BLK,),
        in_specs=[pl.BlockSpec((_N, _D), lambda i: (0, 0))],
        out_specs=pl.BlockSpec((1, 1), lambda i: (0, 0)),
        out_shape=jax.ShapeDtypeStruct((1, 1), jnp.float32),
        compiler_params=pltpu.CompilerParams(
            dimension_semantics=("arbitrary",),
        ),
    )(z)
    return -partial[0, 0] / _N


# speedup vs baseline: 54.1424x; 1.9460x over previous
import jax
import jax.numpy as jnp
from jax.experimental import pallas as pl
from jax.experimental.pallas import tpu as pltpu

_T = 0.1
_K = 5
_N = 4096
_D = 128
_BLK = 256
_LANES = 128
_NCAND = (_K + 1) * _LANES


def _and_loss_kernel(z_ref, out_ref, zn_ref):
    i = pl.program_id(0)

    @pl.when(i == 0)
    def _():
        z = z_ref[...]
        nrm = jnp.sqrt(jnp.sum(z * z, axis=1, keepdims=True))
        zn_ref[...] = (z / jnp.maximum(nrm, 1e-12)) * (_T ** -0.5)

    zn = zn_ref[...]
    zb = zn_ref[pl.ds(i * _BLK, _BLK), :]
    logits = jnp.dot(zb, zn.T, preferred_element_type=jnp.float32)

    t = [jnp.full((_BLK, _LANES), -jnp.inf, jnp.float32) for _ in range(_K + 1)]
    for c in range(_N // _LANES):
        x = logits[:, c * _LANES:(c + 1) * _LANES]
        for j in range(_K + 1):
            hi = jnp.maximum(t[j], x)
            x = jnp.minimum(t[j], x)
            t[j] = hi

    m0 = jnp.max(t[0], axis=1, keepdims=True)

    s_full = jnp.sum(jnp.exp(logits - m0), axis=1, keepdims=True)
    lse = m0 + jnp.log(s_full - 1.0)

    cand = jnp.concatenate(t, axis=1)
    col = jax.lax.broadcasted_iota(jnp.int32, (_BLK, _NCAND), 1)
    cur = cand
    vals = []
    for _ in range(_K + 1):
        vals.append(jnp.max(cur, axis=1, keepdims=True))
        am = jnp.argmax(cur, axis=1)
        cur = jnp.where(col == am[:, None], -jnp.inf, cur)

    acc = jnp.ones_like(vals[1])
    for j in range(2, _K + 1):
        acc = acc + jnp.exp(vals[j] - vals[1])
    num = vals[1] + jnp.log(acc)

    part = jnp.sum(num - lse, keepdims=True)

    @pl.when(i == 0)
    def _():
        out_ref[...] = jnp.zeros((1, 1), jnp.float32)

    out_ref[...] += part


def kernel(z):
    partial = pl.pallas_call(
        _and_loss_kernel,
        grid=(_N // _BLK,),
        in_specs=[pl.BlockSpec((_N, _D), lambda i: (0, 0))],
        out_specs=pl.BlockSpec((1, 1), lambda i: (0, 0)),
        out_shape=jax.ShapeDtypeStruct((1, 1), jnp.float32),
        scratch_shapes=[pltpu.VMEM((_N, _D), jnp.float32)],
        compiler_params=pltpu.CompilerParams(
            dimension_semantics=("arbitrary",),
        ),
    )(z)
    return -partial[0, 0] / _N


# speedup vs baseline: 66.7315x; 1.2325x over previous
import jax
import jax.numpy as jnp
from jax.experimental import pallas as pl
from jax.experimental.pallas import tpu as pltpu

_T = 0.1
_K = 5
_N = 4096
_D = 128
_BLK = 256
_LANES = 128
_M = 1.0 / _T


def _and_loss_kernel(z_ref, out_ref, zn_ref):
    i = pl.program_id(0)

    @pl.when(i == 0)
    def _():
        z = z_ref[...]
        nrm = jnp.sqrt(jnp.sum(z * z, axis=1, keepdims=True))
        zn_ref[...] = (z / jnp.maximum(nrm, 1e-12)) * (_T ** -0.5)

    zn = zn_ref[...]
    zb = zn_ref[pl.ds(i * _BLK, _BLK), :]
    logits = jnp.dot(zb, zn.T, preferred_element_type=jnp.float32)

    neg = jnp.float32(-jnp.inf)
    t = [jnp.full((_BLK, _LANES), neg, jnp.float32) for _ in range(_K + 1)]
    e_acc = jnp.zeros((_BLK, _LANES), jnp.float32)
    for c in range(_N // _LANES):
        x = logits[:, c * _LANES:(c + 1) * _LANES]
        e_acc = e_acc + jnp.exp(x - _M)
        for j in range(_K + 1):
            hi = jnp.maximum(t[j], x)
            x = jnp.minimum(t[j], x)
            t[j] = hi

    m0 = jnp.max(t[0], axis=1, keepdims=True)
    s_full = jnp.sum(e_acc, axis=1, keepdims=True)
    lse = _M + jnp.log(s_full - jnp.exp(m0 - _M))

    mask0 = t[0] == m0
    u = [jnp.where(mask0, t[j + 1], t[j]) for j in range(_K)]

    vals = []
    for k in range(_K):
        vk = jnp.max(u[0], axis=1, keepdims=True)
        vals.append(vk)
        if k < _K - 1:
            mk = u[0] == vk
            u = [jnp.where(mk, u[j + 1], u[j]) for j in range(_K - 1)] + [
                jnp.where(mk, neg, u[_K - 1])
            ]

    acc = jnp.ones_like(vals[0])
    for k in range(1, _K):
        acc = acc + jnp.exp(vals[k] - vals[0])
    num = vals[0] + jnp.log(acc)

    part = jnp.sum(num - lse, keepdims=True)

    @pl.when(i == 0)
    def _():
        out_ref[...] = jnp.zeros((1, 1), jnp.float32)

    out_ref[...] += part


def kernel(z):
    partial = pl.pallas_call(
        _and_loss_kernel,
        grid=(_N // _BLK,),
        in_specs=[pl.BlockSpec((_N, _D), lambda i: (0, 0))],
        out_specs=pl.BlockSpec((1, 1), lambda i: (0, 0)),
        out_shape=jax.ShapeDtypeStruct((1, 1), jnp.float32),
        scratch_shapes=[pltpu.VMEM((_N, _D), jnp.float32)],
        compiler_params=pltpu.CompilerParams(
            dimension_semantics=("arbitrary",),
        ),
    )(z)
    return -partial[0, 0] / _N
